# baseline (speedup 1.0000x reference)
import math

import jax
import jax.numpy as jnp
from jax import lax
from jax.experimental import pallas as pl
from jax.experimental.pallas import tpu as pltpu
from jax.experimental.pallas import tpu_sc as plsc

_N = 10000
_D = 128
_E = 320000
_G = 64

_NC = 2
_NS = 16
_NW = _NC * _NS
_EPW = _E // _NW
_K = 128
_NCH = _E // _K // _NW
_NTAIL = _E // _K - _NCH * _NW
_RSTEP = 624
_RPT = 640


def _agg_body(x_hbm, ei_hbm, out_hbm, idxb, rows0, rows1, rows2, acc,
              semi0, semi1, semi2, semg0, semg1, semg2):
    c = lax.axis_index("c")
    s = lax.axis_index("s")
    wid = s * _NC + c
    cg0 = wid * _NCH

    def _z(i, carry):
        rows0[i // 8, pl.ds((i % 8) * 16, 16)] = jnp.zeros((16,), jnp.float32)
        return carry

    lax.fori_loop(0, _K * (_D // 16), _z, 0)
    base = s * _RSTEP
    for t in range(_RPT // _K):
        pltpu.sync_copy(rows0, acc.at[pl.ds(base + t * _K, _K)])
    plsc.subcore_barrier()

    bufs = (rows0, rows1, rows2)
    semi = (semi0, semi1, semi2)
    semg = (semg0, semg1, semg2)

    def _ei(j):
        return ei_hbm.at[pl.ds(0, 2), pl.ds((cg0 + j) * _K, _K)]

    def _issue_idx(j, p):
        pltpu.async_copy(_ei(j), idxb.at[pl.ds(2 * p, 2)], semi[p])

    def _wait_idx(j, p):
        pltpu.make_async_copy(_ei(j), idxb.at[pl.ds(2 * p, 2)],
                              semi[p]).wait()

    def _issue_g(j, p):
        pltpu.async_copy(x_hbm.at[idxb.at[2 * p]], bufs[p], semg[p])

    def _scatter(j, p):
        pltpu.make_async_copy(x_hbm.at[idxb.at[2 * p]], bufs[p],
                              semg[p]).wait()
        pltpu.sync_copy(bufs[p], acc.at[idxb.at[2 * p + 1]], add=True)

    _issue_idx(0, 0)
    _issue_idx(1, 1)
    _issue_idx(2, 2)
    _wait_idx(0, 0)
    _issue_g(0, 0)
    _wait_idx(1, 1)
    _issue_g(1, 1)

    def _tri(i, carry):
        j = 3 * i
        for u in range(3):
            m = j + u
            _wait_idx(m + 2, (u + 2) % 3)
            _issue_g(m + 2, (u + 2) % 3)
            _scatter(m, u)
            _issue_idx(m + 3, u)
        return carry

    lax.fori_loop(0, (_NCH - 3) // 3, _tri, 0)
    _wait_idx(_NCH - 1, 2)
    _issue_g(_NCH - 1, 2)
    _scatter(_NCH - 3, 0)
    _scatter(_NCH - 2, 1)
    _scatter(_NCH - 1, 2)

    @pl.when(wid < _NTAIL)
    def _():
        jt = _NW * _NCH - cg0 + wid
        _issue_idx(jt, 0)
        _wait_idx(jt, 0)
        _issue_g(jt, 0)
        _scatter(jt, 0)

    plsc.subcore_barrier()

    pltpu.sync_copy(acc.at[pl.ds(base, _RPT)],
                    out_hbm.at[pl.ds(c * _N + base, _RPT)])


def _aggregate(x, edge_index):
    f = pl.kernel(
        _agg_body,
        out_type=jax.ShapeDtypeStruct((_NC * _N, _D), jnp.float32),
        mesh=plsc.VectorSubcoreMesh(core_axis_name="c", subcore_axis_name="s"),
        scratch_types=[
            pltpu.VMEM((8, _K), jnp.int32),
            pltpu.VMEM((_K, _D), jnp.float32),
            pltpu.VMEM((_K, _D), jnp.float32),
            pltpu.VMEM((_K, _D), jnp.float32),
            pltpu.VMEM_SHARED((_N, _D), jnp.float32),
            pltpu.SemaphoreType.DMA,
            pltpu.SemaphoreType.DMA,
            pltpu.SemaphoreType.DMA,
            pltpu.SemaphoreType.DMA,
            pltpu.SemaphoreType.DMA,
            pltpu.SemaphoreType.DMA,
        ],
    )
    return f(x, edge_index)


def _mlp_body(x_ref, a0_ref, a1_ref, w1_ref, b1_ref, w2_ref, b2_ref,
              sc_ref, sh_ref, o_ref):
    h = x_ref[...] + a0_ref[...] + a1_ref[...]
    h = jnp.dot(h, w1_ref[...], preferred_element_type=jnp.float32) + b1_ref[...]
    h = jnp.maximum(h, 0.0)
    h = jnp.dot(h, w2_ref[...], preferred_element_type=jnp.float32) + b2_ref[...]
    h = jnp.maximum(h, 0.0)
    o_ref[...] = h * sc_ref[...] + sh_ref[...]


def _mlp(x, agg, w1t, b1, w2t, b2, scale, shift):
    br = 1000
    nb = _N // br
    return pl.pallas_call(
        _mlp_body,
        grid=(nb,),
        in_specs=[
            pl.BlockSpec((br, _D), lambda i: (i, 0)),
            pl.BlockSpec((br, _D), lambda i: (i, 0)),
            pl.BlockSpec((br, _D), lambda i: (i + nb, 0)),
            pl.BlockSpec((_D, _D), lambda i: (0, 0)),
            pl.BlockSpec((1, _D), lambda i: (0, 0)),
            pl.BlockSpec((_D, _D), lambda i: (0, 0)),
            pl.BlockSpec((1, _D), lambda i: (0, 0)),
            pl.BlockSpec((1, _D), lambda i: (0, 0)),
            pl.BlockSpec((1, _D), lambda i: (0, 0)),
        ],
        out_specs=pl.BlockSpec((br, _D), lambda i: (i, 0)),
        out_shape=jax.ShapeDtypeStruct((_N, _D), jnp.float32),
    )(x, agg, agg, w1t, b1, w2t, b2, scale, shift)


_BR = 1000
_NB = _N // _BR


def _mlp_pool_body(x_ref, a0_ref, a1_ref, w1_ref, b1_ref, w2_ref, b2_ref,
                   sc_ref, sh_ref, b_ref, w5_ref, b5_ref, w6_ref, b6_ref,
                   o_ref, sums_acc, cnt_acc):
    i = pl.program_id(0)
    hp = None

    h = x_ref[...] + a0_ref[...] + a1_ref[...]
    h = jnp.dot(h, w1_ref[...], preferred_element_type=jnp.float32,
                precision=hp) + b1_ref[...]
    h = jnp.maximum(h, 0.0)
    h = jnp.dot(h, w2_ref[...], preferred_element_type=jnp.float32,
                precision=hp) + b2_ref[...]
    h = jnp.maximum(h, 0.0)
    h = h * sc_ref[...] + sh_ref[...]

    bv = b_ref[0, 0, :]
    oh = (lax.broadcasted_iota(jnp.int32, (_G, _BR), 0) == bv[None, :])
    oh = oh.astype(jnp.float32)
    dn = (((1,), (0,)), ((), ()))
    sums = lax.dot_general(oh, h, dn, preferred_element_type=jnp.float32,
                           precision=hp)
    ones = jnp.ones((_BR, _D), jnp.float32)
    cnt = lax.dot_general(oh, ones, dn, preferred_element_type=jnp.float32,
                          precision=hp)

    @pl.when(i == 0)
    def _():
        sums_acc[...] = sums
        cnt_acc[...] = cnt

    @pl.when(i > 0)
    def _():
        sums_acc[...] += sums
        cnt_acc[...] += cnt

    @pl.when(i == _NB - 1)
    def _():
        pooled = sums_acc[...] / jnp.maximum(cnt_acc[...], 1.0)
        p = jnp.dot(pooled, w5_ref[...], preferred_element_type=jnp.float32,
                    precision=hp) + b5_ref[...]
        p = jnp.maximum(p, 0.0)
        o = jnp.dot(p, w6_ref[...], preferred_element_type=jnp.float32,
                    precision=hp) + b6_ref[...]
        m = jnp.max(o, axis=-1, keepdims=True)
        lse = jnp.log(jnp.sum(jnp.exp(o - m), axis=-1, keepdims=True))
        o_ref[...] = o - m - lse


def _mlp_pool(h1, agg, w3t, b3, w4t, b4, scale, shift, batch2d,
              w5t, b5, w6t, b6):
    full = pl.BlockSpec((_D, _D), lambda i: (0, 0))
    row = pl.BlockSpec((1, _D), lambda i: (0, 0))
    return pl.pallas_call(
        _mlp_pool_body,
        grid=(_NB,),
        in_specs=[
            pl.BlockSpec((_BR, _D), lambda i: (i, 0)),
            pl.BlockSpec((_BR, _D), lambda i: (i, 0)),
            pl.BlockSpec((_BR, _D), lambda i: (i + _NB, 0)),
            full, row, full, row, row, row,
            pl.BlockSpec((1, 1, _BR), lambda i: (i, 0, 0)),
            full, row, full, row,
        ],
        out_specs=pl.BlockSpec((_G, _D), lambda i: (0, 0)),
        out_shape=jax.ShapeDtypeStruct((_G, _D), jnp.float32),
        scratch_shapes=[
            pltpu.VMEM((_G, _D), jnp.float32),
            pltpu.VMEM((_G, _D), jnp.float32),
        ],
    )(h1, agg, agg, w3t, b3, w4t, b4, scale, shift, batch2d,
      w5t, b5, w6t, b6)


def kernel(x, edge_index, batch, W1, b1, W2, b2, g1, be1,
           W3, b3, W4, b4, g2, be2, W5, b5, W6, b6):
    inv = 1.0 / math.sqrt(1.0 + 1e-5)

    agg = _aggregate(x, edge_index)
    h = _mlp(x, agg, W1.T, b1.reshape(1, _D), W2.T, b2.reshape(1, _D),
             (g1 * inv).reshape(1, _D), be1.reshape(1, _D))
    agg = _aggregate(h, edge_index)
    return _mlp_pool(h, agg, W3.T, b3.reshape(1, _D), W4.T, b4.reshape(1, _D),
                     (g2 * inv).reshape(1, _D), be2.reshape(1, _D),
                     batch.reshape(_NB, 1, _BR), W5.T, b5.reshape(1, _D),
                     W6.T, b6.reshape(1, _D))


# speedup vs baseline: 1.1480x; 1.1480x over previous
import math

import jax
import jax.numpy as jnp
from jax import lax
from jax.experimental import pallas as pl
from jax.experimental.pallas import tpu as pltpu
from jax.experimental.pallas import tpu_sc as plsc

_N = 10000
_D = 128
_E = 320000
_G = 64

_NC = 2
_NS = 16
_NW = _NC * _NS
_EPW = _E // _NW
_K = 80
_NCH = _EPW // _K
_RSTEP = 624
_RPT = 640


def _agg_body(x_hbm, src_hbm, dst_hbm, out_hbm, src_v, dstst,
              rows0, rows1, rows2, acc, sem0, sem1, sem2):
    c = lax.axis_index("c")
    s = lax.axis_index("s")
    wid = s * _NC + c

    pltpu.sync_copy(src_hbm.at[pl.ds(wid * _EPW, _EPW)], src_v)

    def _z(i, carry):
        rows0[i // 8, pl.ds((i % 8) * 16, 16)] = jnp.zeros((16,), jnp.float32)
        return carry

    lax.fori_loop(0, _K * (_D // 16), _z, 0)
    base = s * _RSTEP
    for t in range(_RPT // _K):
        pltpu.sync_copy(rows0, acc.at[pl.ds(base + t * _K, _K)])
    plsc.subcore_barrier()

    bufs = (rows0, rows1, rows2)
    sems = (sem0, sem1, sem2)

    def _issue(j, b):
        pltpu.async_copy(dst_hbm.at[pl.ds(wid * _EPW + j * _K, _K)],
                         dstst.at[b], sems[b])
        pltpu.async_copy(x_hbm.at[src_v.at[pl.ds(j * _K, _K)]],
                         bufs[b], sems[b])

    def _drain(j, b):
        pltpu.make_async_copy(dst_hbm.at[pl.ds(wid * _EPW + j * _K, _K)],
                              dstst.at[b], sems[b]).wait()
        pltpu.make_async_copy(x_hbm.at[src_v.at[pl.ds(j * _K, _K)]],
                              bufs[b], sems[b]).wait()
        pltpu.sync_copy(bufs[b], acc.at[dstst.at[b]], add=True)

    _issue(0, 0)
    _issue(1, 1)

    def _tri(i, carry):
        j = 3 * i
        _issue(j + 2, 2)
        _drain(j, 0)
        _issue(j + 3, 0)
        _drain(j + 1, 1)
        _issue(j + 4, 1)
        _drain(j + 2, 2)
        return carry

    lax.fori_loop(0, (_NCH - 2) // 3, _tri, 0)
    _drain(_NCH - 2, (_NCH - 2) % 3)
    _drain(_NCH - 1, (_NCH - 1) % 3)
    plsc.subcore_barrier()

    pltpu.sync_copy(acc.at[pl.ds(base, _RPT)],
                    out_hbm.at[pl.ds(c * _N + base, _RPT)])


def _aggregate(x, src1d, dst1d):
    f = pl.kernel(
        _agg_body,
        out_type=jax.ShapeDtypeStruct((_NC * _N, _D), jnp.float32),
        mesh=plsc.VectorSubcoreMesh(core_axis_name="c", subcore_axis_name="s"),
        scratch_types=[
            pltpu.VMEM((_EPW,), jnp.int32),
            pltpu.VMEM((8, _K), jnp.int32),
            pltpu.VMEM((_K, _D), jnp.float32),
            pltpu.VMEM((_K, _D), jnp.float32),
            pltpu.VMEM((_K, _D), jnp.float32),
            pltpu.VMEM_SHARED((_N, _D), jnp.float32),
            pltpu.SemaphoreType.DMA,
            pltpu.SemaphoreType.DMA,
            pltpu.SemaphoreType.DMA,
        ],
    )
    return f(x, src1d, dst1d)


def _mlp_body(x_ref, a0_ref, a1_ref, w1_ref, b1_ref, w2_ref, b2_ref,
              sc_ref, sh_ref, o_ref):
    h = x_ref[...] + a0_ref[...] + a1_ref[...]
    h = jnp.dot(h, w1_ref[...], preferred_element_type=jnp.float32) + b1_ref[...]
    h = jnp.maximum(h, 0.0)
    h = jnp.dot(h, w2_ref[...], preferred_element_type=jnp.float32) + b2_ref[...]
    h = jnp.maximum(h, 0.0)
    o_ref[...] = h * sc_ref[...] + sh_ref[...]


def _mlp(x, agg, w1t, b1, w2t, b2, scale, shift):
    br = 1000
    nb = _N // br
    return pl.pallas_call(
        _mlp_body,
        grid=(nb,),
        in_specs=[
            pl.BlockSpec((br, _D), lambda i: (i, 0)),
            pl.BlockSpec((br, _D), lambda i: (i, 0)),
            pl.BlockSpec((br, _D), lambda i: (i + nb, 0)),
            pl.BlockSpec((_D, _D), lambda i: (0, 0)),
            pl.BlockSpec((1, _D), lambda i: (0, 0)),
            pl.BlockSpec((_D, _D), lambda i: (0, 0)),
            pl.BlockSpec((1, _D), lambda i: (0, 0)),
            pl.BlockSpec((1, _D), lambda i: (0, 0)),
            pl.BlockSpec((1, _D), lambda i: (0, 0)),
        ],
        out_specs=pl.BlockSpec((br, _D), lambda i: (i, 0)),
        out_shape=jax.ShapeDtypeStruct((_N, _D), jnp.float32),
    )(x, agg, agg, w1t, b1, w2t, b2, scale, shift)


_BR = 1000
_NB = _N // _BR


def _mlp_pool_body(x_ref, a0_ref, a1_ref, w1_ref, b1_ref, w2_ref, b2_ref,
                   sc_ref, sh_ref, b_ref, w5_ref, b5_ref, w6_ref, b6_ref,
                   o_ref, sums_acc, cnt_acc):
    i = pl.program_id(0)
    hp = None

    h = x_ref[...] + a0_ref[...] + a1_ref[...]
    h = jnp.dot(h, w1_ref[...], preferred_element_type=jnp.float32,
                precision=hp) + b1_ref[...]
    h = jnp.maximum(h, 0.0)
    h = jnp.dot(h, w2_ref[...], preferred_element_type=jnp.float32,
                precision=hp) + b2_ref[...]
    h = jnp.maximum(h, 0.0)
    h = h * sc_ref[...] + sh_ref[...]

    bv = b_ref[0, 0, :]
    oh = (lax.broadcasted_iota(jnp.int32, (_G, _BR), 0) == bv[None, :])
    oh = oh.astype(jnp.float32)
    dn = (((1,), (0,)), ((), ()))
    sums = lax.dot_general(oh, h, dn, preferred_element_type=jnp.float32,
                           precision=hp)
    ones = jnp.ones((_BR, _D), jnp.float32)
    cnt = lax.dot_general(oh, ones, dn, preferred_element_type=jnp.float32,
                          precision=hp)

    @pl.when(i == 0)
    def _():
        sums_acc[...] = sums
        cnt_acc[...] = cnt

    @pl.when(i > 0)
    def _():
        sums_acc[...] += sums
        cnt_acc[...] += cnt

    @pl.when(i == _NB - 1)
    def _():
        pooled = sums_acc[...] / jnp.maximum(cnt_acc[...], 1.0)
        p = jnp.dot(pooled, w5_ref[...], preferred_element_type=jnp.float32,
                    precision=hp) + b5_ref[...]
        p = jnp.maximum(p, 0.0)
        o = jnp.dot(p, w6_ref[...], preferred_element_type=jnp.float32,
                    precision=hp) + b6_ref[...]
        m = jnp.max(o, axis=-1, keepdims=True)
        lse = jnp.log(jnp.sum(jnp.exp(o - m), axis=-1, keepdims=True))
        o_ref[...] = o - m - lse


def _mlp_pool(h1, agg, w3t, b3, w4t, b4, scale, shift, batch2d,
              w5t, b5, w6t, b6):
    full = pl.BlockSpec((_D, _D), lambda i: (0, 0))
    row = pl.BlockSpec((1, _D), lambda i: (0, 0))
    return pl.pallas_call(
        _mlp_pool_body,
        grid=(_NB,),
        in_specs=[
            pl.BlockSpec((_BR, _D), lambda i: (i, 0)),
            pl.BlockSpec((_BR, _D), lambda i: (i, 0)),
            pl.BlockSpec((_BR, _D), lambda i: (i + _NB, 0)),
            full, row, full, row, row, row,
            pl.BlockSpec((1, 1, _BR), lambda i: (i, 0, 0)),
            full, row, full, row,
        ],
        out_specs=pl.BlockSpec((_G, _D), lambda i: (0, 0)),
        out_shape=jax.ShapeDtypeStruct((_G, _D), jnp.float32),
        scratch_shapes=[
            pltpu.VMEM((_G, _D), jnp.float32),
            pltpu.VMEM((_G, _D), jnp.float32),
        ],
    )(h1, agg, agg, w3t, b3, w4t, b4, scale, shift, batch2d,
      w5t, b5, w6t, b6)


def kernel(x, edge_index, batch, W1, b1, W2, b2, g1, be1,
           W3, b3, W4, b4, g2, be2, W5, b5, W6, b6):
    src1d = edge_index[0]
    dst1d = edge_index[1]
    inv = 1.0 / math.sqrt(1.0 + 1e-5)

    agg = _aggregate(x, src1d, dst1d)
    h = _mlp(x, agg, W1.T, b1.reshape(1, _D), W2.T, b2.reshape(1, _D),
             (g1 * inv).reshape(1, _D), be1.reshape(1, _D))
    agg = _aggregate(h, src1d, dst1d)
    return _mlp_pool(h, agg, W3.T, b3.reshape(1, _D), W4.T, b4.reshape(1, _D),
                     (g2 * inv).reshape(1, _D), be2.reshape(1, _D),
                     batch.reshape(_NB, 1, _BR), W5.T, b5.reshape(1, _D),
                     W6.T, b6.reshape(1, _D))


# speedup vs baseline: 1.1766x; 1.0249x over previous
import math

import jax
import jax.numpy as jnp
from jax import lax
from jax.experimental import pallas as pl
from jax.experimental.pallas import tpu as pltpu
from jax.experimental.pallas import tpu_sc as plsc

_N = 10000
_D = 128
_E = 320000
_G = 64

_NC = 2
_NS = 16
_NW = _NC * _NS
_EPW = _E // _NW
_K = 80
_NCH = _EPW // _K
_RSTEP = 624
_RPT = 640


def _agg_body(x_hbm, src_hbm, dst_hbm, out_hbm, src_v, dstst,
              rows0, rows1, rows2, acc, sem0, sem1, sem2):
    c = lax.axis_index("c")
    s = lax.axis_index("s")
    wid = s * _NC + c

    pltpu.async_copy(src_hbm.at[pl.ds(wid * _EPW, _EPW)], src_v, sem0)

    def _z(i, carry):
        rows0[i // 8, pl.ds((i % 8) * 16, 16)] = jnp.zeros((16,), jnp.float32)
        return carry

    lax.fori_loop(0, _K * (_D // 16), _z, 0)
    pltpu.make_async_copy(src_hbm.at[pl.ds(wid * _EPW, _EPW)], src_v,
                          sem0).wait()

    bufs = (rows0, rows1, rows2)
    sems = (sem0, sem1, sem2)

    def _issue(j, b):
        pltpu.async_copy(dst_hbm.at[pl.ds(wid * _EPW + j * _K, _K)],
                         dstst.at[b], sems[b])
        pltpu.async_copy(x_hbm.at[src_v.at[pl.ds(j * _K, _K)]],
                         bufs[b], sems[b])

    def _drain(j, b):
        pltpu.make_async_copy(dst_hbm.at[pl.ds(wid * _EPW + j * _K, _K)],
                              dstst.at[b], sems[b]).wait()
        pltpu.make_async_copy(x_hbm.at[src_v.at[pl.ds(j * _K, _K)]],
                              bufs[b], sems[b]).wait()
        pltpu.sync_copy(bufs[b], acc.at[dstst.at[b]], add=True)

    _issue(0, 1)
    _issue(1, 2)
    base = s * _RSTEP
    for t in range(_RPT // _K):
        pltpu.sync_copy(rows0, acc.at[pl.ds(base + t * _K, _K)])
    plsc.subcore_barrier()

    def _tri(i, carry):
        j = 3 * i
        _issue(j + 2, 0)
        _drain(j, 1)
        _issue(j + 3, 1)
        _drain(j + 1, 2)
        _issue(j + 4, 2)
        _drain(j + 2, 0)
        return carry

    lax.fori_loop(0, (_NCH - 2) // 3, _tri, 0)
    _drain(_NCH - 2, (_NCH - 1) % 3)
    _drain(_NCH - 1, _NCH % 3)
    plsc.subcore_barrier()

    pltpu.sync_copy(acc.at[pl.ds(base, _RPT)],
                    out_hbm.at[pl.ds(c * _N + base, _RPT)])


def _aggregate(x, src1d, dst1d):
    f = pl.kernel(
        _agg_body,
        out_type=jax.ShapeDtypeStruct((_NC * _N, _D), jnp.float32),
        mesh=plsc.VectorSubcoreMesh(core_axis_name="c", subcore_axis_name="s"),
        scratch_types=[
            pltpu.VMEM((_EPW,), jnp.int32),
            pltpu.VMEM((8, _K), jnp.int32),
            pltpu.VMEM((_K, _D), jnp.float32),
            pltpu.VMEM((_K, _D), jnp.float32),
            pltpu.VMEM((_K, _D), jnp.float32),
            pltpu.VMEM_SHARED((_N, _D), jnp.float32),
            pltpu.SemaphoreType.DMA,
            pltpu.SemaphoreType.DMA,
            pltpu.SemaphoreType.DMA,
        ],
    )
    return f(x, src1d, dst1d)


def _mlp_body(x_ref, a0_ref, a1_ref, w1_ref, b1_ref, w2_ref, b2_ref,
              sc_ref, sh_ref, o_ref):
    h = x_ref[...] + a0_ref[...] + a1_ref[...]
    h = jnp.dot(h, w1_ref[...], preferred_element_type=jnp.float32) + b1_ref[...]
    h = jnp.maximum(h, 0.0)
    h = jnp.dot(h, w2_ref[...], preferred_element_type=jnp.float32) + b2_ref[...]
    h = jnp.maximum(h, 0.0)
    o_ref[...] = h * sc_ref[...] + sh_ref[...]


def _mlp(x, agg, w1t, b1, w2t, b2, scale, shift):
    br = 1000
    nb = _N // br
    return pl.pallas_call(
        _mlp_body,
        grid=(nb,),
        in_specs=[
            pl.BlockSpec((br, _D), lambda i: (i, 0)),
            pl.BlockSpec((br, _D), lambda i: (i, 0)),
            pl.BlockSpec((br, _D), lambda i: (i + nb, 0)),
            pl.BlockSpec((_D, _D), lambda i: (0, 0)),
            pl.BlockSpec((1, _D), lambda i: (0, 0)),
            pl.BlockSpec((_D, _D), lambda i: (0, 0)),
            pl.BlockSpec((1, _D), lambda i: (0, 0)),
            pl.BlockSpec((1, _D), lambda i: (0, 0)),
            pl.BlockSpec((1, _D), lambda i: (0, 0)),
        ],
        out_specs=pl.BlockSpec((br, _D), lambda i: (i, 0)),
        out_shape=jax.ShapeDtypeStruct((_N, _D), jnp.float32),
    )(x, agg, agg, w1t, b1, w2t, b2, scale, shift)


_BR = 1000
_NB = _N // _BR


def _mlp_pool_body(x_ref, a0_ref, a1_ref, w1_ref, b1_ref, w2_ref, b2_ref,
                   sc_ref, sh_ref, b_ref, w5_ref, b5_ref, w6_ref, b6_ref,
                   o_ref, sums_acc, cnt_acc):
    i = pl.program_id(0)
    hp = None

    h = x_ref[...] + a0_ref[...] + a1_ref[...]
    h = jnp.dot(h, w1_ref[...], preferred_element_type=jnp.float32,
                precision=hp) + b1_ref[...]
    h = jnp.maximum(h, 0.0)
    h = jnp.dot(h, w2_ref[...], preferred_element_type=jnp.float32,
                precision=hp) + b2_ref[...]
    h = jnp.maximum(h, 0.0)
    h = h * sc_ref[...] + sh_ref[...]

    bv = b_ref[0, 0, :]
    oh = (lax.broadcasted_iota(jnp.int32, (_G, _BR), 0) == bv[None, :])
    oh = oh.astype(jnp.float32)
    dn = (((1,), (0,)), ((), ()))
    sums = lax.dot_general(oh, h, dn, preferred_element_type=jnp.float32,
                           precision=hp)
    ones = jnp.ones((_BR, _D), jnp.float32)
    cnt = lax.dot_general(oh, ones, dn, preferred_element_type=jnp.float32,
                          precision=hp)

    @pl.when(i == 0)
    def _():
        sums_acc[...] = sums
        cnt_acc[...] = cnt

    @pl.when(i > 0)
    def _():
        sums_acc[...] += sums
        cnt_acc[...] += cnt

    @pl.when(i == _NB - 1)
    def _():
        pooled = sums_acc[...] / jnp.maximum(cnt_acc[...], 1.0)
        p = jnp.dot(pooled, w5_ref[...], preferred_element_type=jnp.float32,
                    precision=hp) + b5_ref[...]
        p = jnp.maximum(p, 0.0)
        o = jnp.dot(p, w6_ref[...], preferred_element_type=jnp.float32,
                    precision=hp) + b6_ref[...]
        m = jnp.max(o, axis=-1, keepdims=True)
        lse = jnp.log(jnp.sum(jnp.exp(o - m), axis=-1, keepdims=True))
        o_ref[...] = o - m - lse


def _mlp_pool(h1, agg, w3t, b3, w4t, b4, scale, shift, batch2d,
              w5t, b5, w6t, b6):
    full = pl.BlockSpec((_D, _D), lambda i: (0, 0))
    row = pl.BlockSpec((1, _D), lambda i: (0, 0))
    return pl.pallas_call(
        _mlp_pool_body,
        grid=(_NB,),
        in_specs=[
            pl.BlockSpec((_BR, _D), lambda i: (i, 0)),
            pl.BlockSpec((_BR, _D), lambda i: (i, 0)),
            pl.BlockSpec((_BR, _D), lambda i: (i + _NB, 0)),
            full, row, full, row, row, row,
            pl.BlockSpec((1, 1, _BR), lambda i: (i, 0, 0)),
            full, row, full, row,
        ],
        out_specs=pl.BlockSpec((_G, _D), lambda i: (0, 0)),
        out_shape=jax.ShapeDtypeStruct((_G, _D), jnp.float32),
        scratch_shapes=[
            pltpu.VMEM((_G, _D), jnp.float32),
            pltpu.VMEM((_G, _D), jnp.float32),
        ],
    )(h1, agg, agg, w3t, b3, w4t, b4, scale, shift, batch2d,
      w5t, b5, w6t, b6)


def kernel(x, edge_index, batch, W1, b1, W2, b2, g1, be1,
           W3, b3, W4, b4, g2, be2, W5, b5, W6, b6):
    src1d = edge_index[0]
    dst1d = edge_index[1]
    inv = 1.0 / math.sqrt(1.0 + 1e-5)

    agg = _aggregate(x, src1d, dst1d)
    h = _mlp(x, agg, W1.T, b1.reshape(1, _D), W2.T, b2.reshape(1, _D),
             (g1 * inv).reshape(1, _D), be1.reshape(1, _D))
    agg = _aggregate(h, src1d, dst1d)
    return _mlp_pool(h, agg, W3.T, b3.reshape(1, _D), W4.T, b4.reshape(1, _D),
                     (g2 * inv).reshape(1, _D), be2.reshape(1, _D),
                     batch.reshape(_NB, 1, _BR), W5.T, b5.reshape(1, _D),
                     W6.T, b6.reshape(1, _D))
